# baseline (speedup 1.0000x reference)
import jax
import jax.numpy as jnp
from jax import lax
from jax.experimental import pallas as pl
from jax.experimental.pallas import tpu as pltpu
from jax.experimental.pallas import tpu_sc as plsc

_NF = 26
_V = 100000
_CV = 366
_ED = 32
_CD = 16
_B = 1024
_T = 50
_IN = 16
_ROWS = _T * _B
_TILE = 64
_NT = _ROWS // _TILE
_TPB = _B // _TILE
_EMBW = _NF * _ED
_OUT_D = _IN + _EMBW + 2 * _CD

_info = plsc.get_sparse_core_info()
_NC = _info.num_cores
_NS = _info.num_subcores
_NW = _NC * _NS
_NTW = _NT // _NW
_L = 16


def _sc_body(x_hbm, xe_hbm, xc_hbm, tab_hbm, cal_hbm, out_hbm,
             idx2_v, cidx2_v, idx_v, cidx_v, gat0, gat1, cgat0, cgat1, x_v,
             isem, gsem, wsem0, wsem1):
    wid = lax.axis_index("s") * _NC + lax.axis_index("c")
    gats = (gat0, gat1)
    cgats = (cgat0, cgat1)

    rows16 = lax.iota(jnp.int32, _L)

    def fire(k, b, wsem):
        tile = wid + k * _NW
        r0 = tile * _TILE
        t = tile // _TPB
        b0 = (tile % _TPB) * _TILE
        iload = pltpu.async_copy(xe_hbm.at[pl.ds(b0, _TILE), t, :],
                                 idx2_v.at[b], isem)
        cload = pltpu.async_copy(xc_hbm.at[pl.ds(b0, _TILE), t, :],
                                 cidx2_v.at[b], isem)
        pltpu.sync_copy(x_hbm.at[pl.ds(b0, _TILE), t, :], x_v.at[b])
        pltpu.async_copy(x_v.at[b],
                         out_hbm.at[pl.ds(r0, _TILE), pl.ds(0, _IN)], wsem)
        iload.wait()
        for f in range(_NF):
            col = jnp.full((_L,), f, jnp.int32)
            for j in range(_TILE // _L):
                vals = plsc.load_gather(idx2_v.at[b], [rows16 + j * _L, col])
                idx_v[b, f, pl.ds(j * _L, _L)] = vals + f * _V
        cload.wait()
        for c in range(2):
            col = jnp.full((_L,), c, jnp.int32)
            for j in range(_TILE // _L):
                vals = plsc.load_gather(cidx2_v.at[b], [rows16 + j * _L, col])
                cidx_v[b, c, pl.ds(j * _L, _L)] = vals
        gathers = []
        for f in range(_NF):
            gathers.append(pltpu.async_copy(
                tab_hbm.at[idx_v.at[b, f]], gats[b].at[f], gsem))
        cgathers = []
        for c in range(2):
            cgathers.append(pltpu.async_copy(
                cal_hbm.at[cidx_v.at[b, c]], cgats[b].at[c], gsem))
        for f in range(_NF):
            gathers[f].wait()
            pltpu.async_copy(
                gats[b].at[f],
                out_hbm.at[pl.ds(r0, _TILE), pl.ds(_IN + f * _ED, _ED)], wsem)
        for c in range(2):
            cgathers[c].wait()
            pltpu.async_copy(
                cgats[b].at[c],
                out_hbm.at[pl.ds(r0, _TILE),
                           pl.ds(_IN + _EMBW + c * _CD, _CD)], wsem)

    def drain(b, wsem):
        pltpu.make_async_copy(out_hbm.at[pl.ds(0, _TILE), pl.ds(0, _IN)],
                              x_v.at[b], wsem).wait()
        for f in range(_NF):
            pltpu.make_async_copy(
                out_hbm.at[pl.ds(0, _TILE), pl.ds(_IN + f * _ED, _ED)],
                gats[b].at[f], wsem).wait()
        for c in range(2):
            pltpu.make_async_copy(
                out_hbm.at[pl.ds(0, _TILE), pl.ds(_IN + _EMBW + c * _CD, _CD)],
                cgats[b].at[c], wsem).wait()

    fire(0, 0, wsem0)
    fire(1, 1, wsem1)

    def body(i, carry):
        k0 = 2 * i + 2
        drain(0, wsem0)
        fire(k0, 0, wsem0)
        drain(1, wsem1)
        fire(k0 + 1, 1, wsem1)
        return carry

    lax.fori_loop(0, (_NTW - 3) // 2, body, 0)
    drain(0, wsem0)
    fire(_NTW - 1, 0, wsem0)
    drain(1, wsem1)
    drain(0, wsem0)


_mesh = plsc.VectorSubcoreMesh(core_axis_name="c", subcore_axis_name="s")

_sc_call = pl.kernel(
    _sc_body,
    out_type=jax.ShapeDtypeStruct((_ROWS, _OUT_D), jnp.float32),
    mesh=_mesh,
    compiler_params=pltpu.CompilerParams(use_tc_tiling_on_sc=False,
                                         needs_layout_passes=False),
    scratch_types=[
        pltpu.VMEM((2, _TILE, _NF), jnp.int32),
        pltpu.VMEM((2, _TILE, 2), jnp.int32),
        pltpu.VMEM((2, _NF, _TILE), jnp.int32),
        pltpu.VMEM((2, 2, _TILE), jnp.int32),
        pltpu.VMEM((_NF, _TILE, _ED), jnp.float32),
        pltpu.VMEM((_NF, _TILE, _ED), jnp.float32),
        pltpu.VMEM((2, _TILE, _CD), jnp.float32),
        pltpu.VMEM((2, _TILE, _CD), jnp.float32),
        pltpu.VMEM((2, _TILE, _IN), jnp.float32),
        pltpu.SemaphoreType.DMA,
        pltpu.SemaphoreType.DMA,
        pltpu.SemaphoreType.DMA,
        pltpu.SemaphoreType.DMA,
    ],
)


def kernel(x, x_emb, x_cal_emb, tables, cal_table):
    tab = tables.reshape(_NF * _V, _ED)
    out = _sc_call(x, x_emb.astype(jnp.int32), x_cal_emb.astype(jnp.int32),
                   tab, cal_table)
    return out.reshape(_T, _B, _OUT_D)


# speedup vs baseline: 1.0565x; 1.0565x over previous
import jax
import jax.numpy as jnp
from jax import lax
from jax.experimental import pallas as pl
from jax.experimental.pallas import tpu as pltpu
from jax.experimental.pallas import tpu_sc as plsc

_NF = 26
_V = 100000
_CV = 366
_ED = 32
_CD = 16
_B = 1024
_T = 50
_IN = 16
_ROWS = _T * _B
_TILE = 64
_NT = _ROWS // _TILE
_TPB = _B // _TILE
_EMBW = _NF * _ED
_OUT_D = _IN + _EMBW + 2 * _CD

_info = plsc.get_sparse_core_info()
_NC = _info.num_cores
_NS = _info.num_subcores
_NW = _NC * _NS
_NTW = _NT // _NW


def _sc_body(x_hbm, xe_hbm, xc_hbm, tab_hbm, cal_hbm, out_hbm,
             idx_v, cidx_v, gat0, gat1, cgat0, cgat1, x_v,
             gsem, wsem0, wsem1):
    wid = lax.axis_index("s") * _NC + lax.axis_index("c")
    gats = (gat0, gat1)
    cgats = (cgat0, cgat1)

    def fire(k, b, wsem):
        tile = wid + k * _NW
        r0 = tile * _TILE
        t = tile // _TPB
        b0 = (tile % _TPB) * _TILE
        pltpu.sync_copy(xe_hbm.at[:, t, pl.ds(b0, _TILE)], idx_v.at[b])
        pltpu.sync_copy(xc_hbm.at[:, t, pl.ds(b0, _TILE)], cidx_v.at[b])
        gathers = [pltpu.async_copy(tab_hbm.at[f].at[idx_v.at[b, f]],
                                    gats[b].at[f], gsem)
                   for f in range(_NF)]
        g_cal = [pltpu.async_copy(cal_hbm.at[cidx_v.at[b, c]],
                                  cgats[b].at[c], gsem)
                 for c in range(2)]
        pltpu.sync_copy(x_hbm.at[pl.ds(b0, _TILE), t, :], x_v.at[b])
        pltpu.async_copy(x_v.at[b],
                         out_hbm.at[pl.ds(r0, _TILE), pl.ds(0, _IN)], wsem)
        for f in range(_NF):
            gathers[f].wait()
            pltpu.async_copy(
                gats[b].at[f],
                out_hbm.at[pl.ds(r0, _TILE), pl.ds(_IN + f * _ED, _ED)], wsem)
        for c in range(2):
            g_cal[c].wait()
            pltpu.async_copy(
                cgats[b].at[c],
                out_hbm.at[pl.ds(r0, _TILE),
                           pl.ds(_IN + _EMBW + c * _CD, _CD)], wsem)

    def drain(b, wsem):
        pltpu.make_async_copy(out_hbm.at[pl.ds(0, _TILE), pl.ds(0, _IN)],
                              x_v.at[b], wsem).wait()
        for f in range(_NF):
            pltpu.make_async_copy(
                out_hbm.at[pl.ds(0, _TILE), pl.ds(_IN + f * _ED, _ED)],
                gats[b].at[f], wsem).wait()
        for c in range(2):
            pltpu.make_async_copy(
                out_hbm.at[pl.ds(0, _TILE), pl.ds(_IN + _EMBW + c * _CD, _CD)],
                cgats[b].at[c], wsem).wait()

    fire(0, 0, wsem0)
    fire(1, 1, wsem1)

    def body(i, carry):
        k0 = 2 * i + 2
        drain(0, wsem0)
        fire(k0, 0, wsem0)
        drain(1, wsem1)
        fire(k0 + 1, 1, wsem1)
        return carry

    lax.fori_loop(0, (_NTW - 3) // 2, body, 0)
    drain(0, wsem0)
    fire(_NTW - 1, 0, wsem0)
    drain(1, wsem1)
    drain(0, wsem0)


_mesh = plsc.VectorSubcoreMesh(core_axis_name="c", subcore_axis_name="s")

_sc_call = pl.kernel(
    _sc_body,
    out_type=jax.ShapeDtypeStruct((_ROWS, _OUT_D), jnp.float32),
    mesh=_mesh,
    compiler_params=pltpu.CompilerParams(use_tc_tiling_on_sc=False,
                                         needs_layout_passes=False),
    scratch_types=[
        pltpu.VMEM((2, _NF, _TILE), jnp.int32),
        pltpu.VMEM((2, 2, _TILE), jnp.int32),
        pltpu.VMEM((_NF, _TILE, _ED), jnp.float32),
        pltpu.VMEM((_NF, _TILE, _ED), jnp.float32),
        pltpu.VMEM((2, _TILE, _CD), jnp.float32),
        pltpu.VMEM((2, _TILE, _CD), jnp.float32),
        pltpu.VMEM((2, _TILE, _IN), jnp.float32),
        pltpu.SemaphoreType.DMA,
        pltpu.SemaphoreType.DMA,
        pltpu.SemaphoreType.DMA,
    ],
)


def kernel(x, x_emb, x_cal_emb, tables, cal_table):
    xe = jnp.transpose(x_emb.astype(jnp.int32), (2, 1, 0))
    xc = jnp.transpose(x_cal_emb.astype(jnp.int32), (2, 1, 0))
    out = _sc_call(x, xe, xc, tables, cal_table)
    return out.reshape(_T, _B, _OUT_D)


# speedup vs baseline: 1.3688x; 1.2956x over previous
import jax
import jax.experimental.layout
import jax.numpy as jnp
from jax import lax
from jax.experimental import pallas as pl
from jax.experimental.pallas import tpu as pltpu
from jax.experimental.pallas import tpu_sc as plsc

_NF = 26
_V = 100000
_CV = 366
_ED = 32
_CD = 16
_B = 1024
_T = 50
_IN = 16
_ROWS = _T * _B
_TILE = 64
_NT = _ROWS // _TILE
_TPB = _B // _TILE
_EMBW = _NF * _ED
_OUT_D = _IN + _EMBW + 2 * _CD

_info = plsc.get_sparse_core_info()
_NC = _info.num_cores
_NS = _info.num_subcores
_NW = _NC * _NS
_NTW = _NT // _NW


def _sc_body(x_hbm, xe_hbm, xc_hbm, tab_hbm, cal_hbm, out_hbm,
             idx_v, cidx_v, gat0, gat1, cgat0, cgat1, x_v,
             gsem, wsem0, wsem1):
    wid = lax.axis_index("s") * _NC + lax.axis_index("c")
    gats = (gat0, gat1)
    cgats = (cgat0, cgat1)

    def fire(k, b, wsem):
        tile = wid + k * _NW
        r0 = tile * _TILE
        t = tile // _TPB
        b0 = (tile % _TPB) * _TILE
        pltpu.sync_copy(xe_hbm.at[:, t, pl.ds(b0, _TILE)], idx_v.at[b])
        pltpu.sync_copy(xc_hbm.at[:, t, pl.ds(b0, _TILE)], cidx_v.at[b])
        gathers = [pltpu.async_copy(tab_hbm.at[f].at[idx_v.at[b, f]],
                                    gats[b].at[f], gsem)
                   for f in range(_NF)]
        g_cal = [pltpu.async_copy(cal_hbm.at[cidx_v.at[b, c]],
                                  cgats[b].at[c], gsem)
                 for c in range(2)]
        pltpu.sync_copy(x_hbm.at[pl.ds(b0, _TILE), t, :], x_v.at[b])
        pltpu.async_copy(x_v.at[b],
                         out_hbm.at[pl.ds(r0, _TILE), pl.ds(0, _IN)], wsem)
        for f in range(_NF):
            gathers[f].wait()
            pltpu.async_copy(
                gats[b].at[f],
                out_hbm.at[pl.ds(r0, _TILE), pl.ds(_IN + f * _ED, _ED)], wsem)
        for c in range(2):
            g_cal[c].wait()
            pltpu.async_copy(
                cgats[b].at[c],
                out_hbm.at[pl.ds(r0, _TILE),
                           pl.ds(_IN + _EMBW + c * _CD, _CD)], wsem)

    def drain(b, wsem):
        pltpu.make_async_copy(out_hbm.at[pl.ds(0, _TILE), pl.ds(0, _IN)],
                              x_v.at[b], wsem).wait()
        for f in range(_NF):
            pltpu.make_async_copy(
                out_hbm.at[pl.ds(0, _TILE), pl.ds(_IN + f * _ED, _ED)],
                gats[b].at[f], wsem).wait()
        for c in range(2):
            pltpu.make_async_copy(
                out_hbm.at[pl.ds(0, _TILE), pl.ds(_IN + _EMBW + c * _CD, _CD)],
                cgats[b].at[c], wsem).wait()

    fire(0, 0, wsem0)
    fire(1, 1, wsem1)

    def body(i, carry):
        k0 = 2 * i + 2
        drain(0, wsem0)
        fire(k0, 0, wsem0)
        drain(1, wsem1)
        fire(k0 + 1, 1, wsem1)
        return carry

    lax.fori_loop(0, (_NTW - 3) // 2, body, 0)
    drain(0, wsem0)
    fire(_NTW - 1, 0, wsem0)
    drain(1, wsem1)
    drain(0, wsem0)


_mesh = plsc.VectorSubcoreMesh(core_axis_name="c", subcore_axis_name="s")

_sc_call = pl.kernel(
    _sc_body,
    out_type=jax.ShapeDtypeStruct((_ROWS, _OUT_D), jnp.float32),
    mesh=_mesh,
    compiler_params=pltpu.CompilerParams(use_tc_tiling_on_sc=False,
                                         needs_layout_passes=False),
    scratch_types=[
        pltpu.VMEM((2, _NF, _TILE), jnp.int32),
        pltpu.VMEM((2, 2, _TILE), jnp.int32),
        pltpu.VMEM((_NF, _TILE, _ED), jnp.float32),
        pltpu.VMEM((_NF, _TILE, _ED), jnp.float32),
        pltpu.VMEM((2, _TILE, _CD), jnp.float32),
        pltpu.VMEM((2, _TILE, _CD), jnp.float32),
        pltpu.VMEM((2, _TILE, _IN), jnp.float32),
        pltpu.SemaphoreType.DMA,
        pltpu.SemaphoreType.DMA,
        pltpu.SemaphoreType.DMA,
    ],
)


def kernel(x, x_emb, x_cal_emb, tables, cal_table):
    xe = jnp.transpose(x_emb.astype(jnp.int32), (2, 1, 0))
    xc = jnp.transpose(x_cal_emb.astype(jnp.int32), (2, 1, 0))
    tab = jax.experimental.layout.with_layout_constraint(
        tables,
        jax.experimental.layout.Layout((0, 1, 2), ((8, 32),)))
    out = _sc_call(x, xe, xc, tab, cal_table)
    return out.reshape(_T, _B, _OUT_D)
